# Initial kernel scaffold; baseline (speedup 1.0000x reference)
import jax
import jax.numpy as jnp
from jax.experimental import pallas as pl


def kernel(x, edge_index, edge_attr, We1, be1, We2, be2, Wn1, bn1, Wn2, bn2):
    raise NotImplementedError("write your pallas kernel here")


# speedup vs baseline: 2.1040x; 2.1040x over previous
import functools

import jax
import jax.numpy as jnp
from jax import lax
from jax.experimental import pallas as pl
from jax.experimental.pallas import tpu as pltpu
from jax.experimental.pallas import tpu_sc as plsc

N = 10000
NP = 10240
E = 320000
D = 128
GW = 144
NC = 2
NS = 16
NW = NC * NS
EPW = E // NW
CH = 40
SUB = 10
NCHUNK = EPW // CH
NSUPER = NCHUNK // SUB
RPT = NP // NS


def _ea_body(ea_ref, c_ref, b_ref, o_ref):
    o_ref[...] = jnp.dot(ea_ref[...], c_ref[...],
                         preferred_element_type=jnp.float32) + b_ref[...]


def _xab_body(x_ref, w_ref, o_ref):
    o_ref[...] = jnp.dot(x_ref[...], w_ref[...],
                         preferred_element_type=jnp.float32)


def _node_body(g0_ref, g1_ref, x_ref, we2e_ref, wn1a_ref, wn1b_ref, bn1_ref,
               wn2_ref, bn2_ref, o_ref):
    g = g0_ref[...] + g1_ref[...]
    agg = jnp.dot(g, we2e_ref[...], preferred_element_type=jnp.float32)
    h2 = jnp.maximum(
        jnp.dot(x_ref[...], wn1a_ref[...], preferred_element_type=jnp.float32)
        + jnp.dot(agg, wn1b_ref[...], preferred_element_type=jnp.float32)
        + bn1_ref[...], 0.0)
    o_ref[...] = jnp.dot(h2, wn2_ref[...],
                         preferred_element_type=jnp.float32) + bn2_ref[...]


def _sc_body(row_hbm, col_hbm, ea_hbm, xa_hbm, xb_hbm, out_hbm,
             row_idx, col_idx, abuf, bbuf, ebuf, hbuf, gsh, sem):
    cid = lax.axis_index("c")
    sid = lax.axis_index("s")
    wid = sid * NC + cid
    base = wid * EPW

    zero16 = jnp.zeros((16,), jnp.float32)
    iota16 = lax.iota(jnp.int32, 16)
    unit16 = jnp.where(iota16 == 0, 1.0, 0.0).astype(jnp.float32)

    def _zstage(i, _):
        r = i // (GW // 16)
        j = i % (GW // 16)
        hbuf[r, pl.ds(j * 16, 16)] = zero16
        return 0
    lax.fori_loop(0, CH * (GW // 16), _zstage, 0)

    def _zcopy(k, _):
        pltpu.sync_copy(hbuf, gsh.at[pl.ds(sid * RPT + k * CH, CH)])
        return 0
    lax.fori_loop(0, RPT // CH, _zcopy, 0)

    def _initcnt(r, _):
        hbuf[r, pl.ds(D, 16)] = unit16
        return 0
    lax.fori_loop(0, CH, _initcnt, 0)

    plsc.subcore_barrier()

    def _super(s, _):
        pltpu.sync_copy(row_hbm.at[pl.ds(wid * NCHUNK + s * SUB, SUB)], row_idx)
        pltpu.sync_copy(col_hbm.at[pl.ds(wid * NCHUNK + s * SUB, SUB)], col_idx)

        def _chunk(j, _):
            off = base + (s * SUB + j) * CH
            cp_a = pltpu.async_copy(xa_hbm.at[row_idx.at[j]], abuf, sem)
            cp_b = pltpu.async_copy(xb_hbm.at[col_idx.at[j]], bbuf, sem)
            cp_e = pltpu.async_copy(ea_hbm.at[pl.ds(off, CH)], ebuf, sem)
            cp_a.wait()
            cp_b.wait()
            cp_e.wait()

            def _elt(i, _):
                r = i // (D // 16)
                jo = (i % (D // 16)) * 16
                a = abuf[r, pl.ds(jo, 16)]
                b = bbuf[r, pl.ds(jo, 16)]
                e = ebuf[r, pl.ds(jo, 16)]
                hbuf[r, pl.ds(jo, 16)] = jnp.maximum(a + b + e, 0.0)
                return 0
            lax.fori_loop(0, CH * (D // 16), _elt, 0)

            pltpu.sync_copy(hbuf, gsh.at[col_idx.at[j]], add=True)
            return 0
        lax.fori_loop(0, SUB, _chunk, 0)
        return 0
    lax.fori_loop(0, NSUPER, _super, 0)

    plsc.subcore_barrier()

    def _wb(k, _):
        r0 = sid * RPT + k * CH
        pltpu.sync_copy(gsh.at[pl.ds(r0, CH)], hbuf)
        pltpu.sync_copy(hbuf, out_hbm.at[cid].at[pl.ds(r0, CH)])
        return 0
    lax.fori_loop(0, RPT // CH, _wb, 0)


_sc_scatter = functools.partial(
    pl.kernel,
    out_type=jax.ShapeDtypeStruct((NC, NP, GW), jnp.float32),
    mesh=plsc.VectorSubcoreMesh(core_axis_name="c", subcore_axis_name="s"),
    compiler_params=pltpu.CompilerParams(use_tc_tiling_on_sc=False),
    scratch_types=[
        pltpu.VMEM((SUB, CH), jnp.int32),
        pltpu.VMEM((SUB, CH), jnp.int32),
        pltpu.VMEM((CH, D), jnp.float32),
        pltpu.VMEM((CH, D), jnp.float32),
        pltpu.VMEM((CH, D), jnp.float32),
        pltpu.VMEM((CH, GW), jnp.float32),
        pltpu.VMEM_SHARED((NP, GW), jnp.float32),
        pltpu.SemaphoreType.DMA,
    ],
)(_sc_body)


def kernel(x, edge_index, edge_attr, We1, be1, We2, be2, Wn1, bn1, Wn2, bn2):
    row = edge_index[0].astype(jnp.int32).reshape(E // CH, CH)
    col = edge_index[1].astype(jnp.int32).reshape(E // CH, CH)

    wab = jnp.concatenate([We1[:D, :], We1[D:2 * D, :]], axis=1)
    wc = We1[2 * D:, :]
    we2e = jnp.zeros((GW, D), jnp.float32).at[:D].set(We2).at[D].set(be2)
    wn1a = Wn1[:D, :]
    wn1b = Wn1[D:, :]

    eb = 2000
    nb = 2000

    xab = pl.pallas_call(
        _xab_body,
        grid=(N // nb,),
        in_specs=[pl.BlockSpec((nb, D), lambda i: (i, 0)),
                  pl.BlockSpec((D, 2 * D), lambda i: (0, 0))],
        out_specs=pl.BlockSpec((nb, 2 * D), lambda i: (i, 0)),
        out_shape=jax.ShapeDtypeStruct((N, 2 * D), jnp.float32),
    )(x, wab)

    ea = pl.pallas_call(
        _ea_body,
        grid=(E // eb,),
        in_specs=[pl.BlockSpec((eb, D), lambda i: (i, 0)),
                  pl.BlockSpec((D, D), lambda i: (0, 0)),
                  pl.BlockSpec((1, D), lambda i: (0, 0))],
        out_specs=pl.BlockSpec((eb, D), lambda i: (i, 0)),
        out_shape=jax.ShapeDtypeStruct((E, D), jnp.float32),
    )(edge_attr, wc, be1.reshape(1, D))

    xa = xab[:, :D]
    xb = xab[:, D:]

    gp = _sc_scatter(row, col, ea, xa, xb)

    new_x = pl.pallas_call(
        _node_body,
        grid=(N // nb,),
        in_specs=[pl.BlockSpec((nb, GW), lambda i: (i, 0)),
                  pl.BlockSpec((nb, GW), lambda i: (i, 0)),
                  pl.BlockSpec((nb, D), lambda i: (i, 0)),
                  pl.BlockSpec((GW, D), lambda i: (0, 0)),
                  pl.BlockSpec((D, D), lambda i: (0, 0)),
                  pl.BlockSpec((D, D), lambda i: (0, 0)),
                  pl.BlockSpec((1, D), lambda i: (0, 0)),
                  pl.BlockSpec((D, D), lambda i: (0, 0)),
                  pl.BlockSpec((1, D), lambda i: (0, 0))],
        out_specs=pl.BlockSpec((nb, D), lambda i: (i, 0)),
        out_shape=jax.ShapeDtypeStruct((N, D), jnp.float32),
    )(gp[0], gp[1], x, we2e, wn1a, wn1b, bn1.reshape(1, D),
      Wn2, bn2.reshape(1, D))

    return new_x
